# baseline (speedup 1.0000x reference)
import functools

import jax
import jax.numpy as jnp
from jax import lax
from jax.experimental import pallas as pl
from jax.experimental.pallas import tpu as pltpu
from jax.experimental.pallas import tpu_sc as plsc

EMB_DIM = 32
NUM_WORKERS = 32
R = 3328


def _emb_body(x_hbm, table_hbm, out_hbm, rows_v, gsem, wsem):
    n_flat = out_hbm.shape[0]
    b_per_w = n_flat // NUM_WORKERS
    wid = lax.axis_index("s") * 2 + lax.axis_index("c")
    base = wid * b_per_w

    n_reads = 4 * (1000000 // 16 // R)
    tbase = (wid % 16) * (1000000 // 16)

    @pl.loop(0, n_reads)
    def _rd(i):
        toff = tbase + (i % 18) * R
        pltpu.async_copy(table_hbm.at[pl.ds(toff, R)], rows_v, gsem).wait()

    @pl.loop(0, b_per_w // R + 1)
    def _wr(c):
        off = base + c * (b_per_w // 4)

        @pl.when(c < 4)
        def _():
            pltpu.async_copy(rows_v, out_hbm.at[pl.ds(off, R)], wsem).wait()


def kernel(x, table):
    batch, n_fields = x.shape
    n_flat = batch * n_fields
    x_flat = x.reshape(n_flat).astype(jnp.int32)

    mesh = plsc.VectorSubcoreMesh(core_axis_name="c", subcore_axis_name="s")
    emb = pl.kernel(
        _emb_body,
        out_type=jax.ShapeDtypeStruct((n_flat, EMB_DIM), jnp.float32),
        mesh=mesh,
        scratch_types=[
            pltpu.VMEM((R, EMB_DIM), jnp.float32),
            pltpu.SemaphoreType.DMA,
            pltpu.SemaphoreType.DMA,
        ],
        compiler_params=pltpu.CompilerParams(use_tc_tiling_on_sc=False),
    )
    out_flat = emb(x_flat, table)
    return out_flat.reshape(batch, n_fields, EMB_DIM)


# speedup vs baseline: 1.4233x; 1.4233x over previous
import functools

import jax
import jax.numpy as jnp
from jax import lax
from jax.experimental import pallas as pl
from jax.experimental.pallas import tpu as pltpu
from jax.experimental.pallas import tpu_sc as plsc

EMB_DIM = 32
NUM_WORKERS = 32
CHUNK = 1024
NBUF = 3


def _emb_body(x_hbm, table_hbm, out_hbm, idx_v, rows0, rows1, rows2, gsem, wsem):
    n_flat = out_hbm.shape[0]
    b_per_w = n_flat // NUM_WORKERS
    n_chunks = b_per_w // CHUNK
    wid = lax.axis_index("s") * 2 + lax.axis_index("c")
    base = wid * b_per_w

    rows = [rows0, rows1, rows2]
    gathers = [None] * n_chunks
    writes = [None] * n_chunks

    pltpu.sync_copy(x_hbm.at[pl.ds(base, b_per_w)], idx_v)

    def start_gather(c):
        return pltpu.async_copy(
            table_hbm.at[idx_v.at[pl.ds(c * CHUNK, CHUNK)]],
            rows[c % NBUF], gsem)

    for c in range(min(NBUF, n_chunks)):
        gathers[c] = start_gather(c)
    for c in range(n_chunks):
        gathers[c].wait()
        writes[c] = pltpu.async_copy(
            rows[c % NBUF], out_hbm.at[pl.ds(base + c * CHUNK, CHUNK)], wsem)
        nxt = c + NBUF
        if nxt < n_chunks:
            writes[nxt - NBUF].wait()
            gathers[nxt] = start_gather(nxt)
    for c in range(max(0, n_chunks - NBUF), n_chunks):
        writes[c].wait()


def kernel(x, table):
    batch, n_fields = x.shape
    n_flat = batch * n_fields
    x_flat = x.reshape(n_flat).astype(jnp.int32)
    b_per_w = n_flat // NUM_WORKERS

    mesh = plsc.VectorSubcoreMesh(core_axis_name="c", subcore_axis_name="s")
    emb = pl.kernel(
        _emb_body,
        out_type=jax.ShapeDtypeStruct((n_flat, EMB_DIM), jnp.float32),
        mesh=mesh,
        scratch_types=[
            pltpu.VMEM((b_per_w,), jnp.int32),
            pltpu.VMEM((CHUNK, EMB_DIM), jnp.float32),
            pltpu.VMEM((CHUNK, EMB_DIM), jnp.float32),
            pltpu.VMEM((CHUNK, EMB_DIM), jnp.float32),
            pltpu.SemaphoreType.DMA,
            pltpu.SemaphoreType.DMA,
        ],
        compiler_params=pltpu.CompilerParams(use_tc_tiling_on_sc=False),
    )
    out_flat = emb(x_flat, table)
    return out_flat.reshape(batch, n_fields, EMB_DIM)
